# Initial kernel scaffold; baseline (speedup 1.0000x reference)
import jax
import jax.numpy as jnp
from jax.experimental import pallas as pl


def kernel(base_question_embeddings, facts_encodings, decoder_hidden, decoder_carry, previous_token_embedding, Wq_k, Wq_h, v_q, Wf_k, Wf_h, v_f, W_lstm, U_lstm, b_lstm, W_r, b_r, U_r, b_u, V_r, b_v, W_y, b_y):
    raise NotImplementedError("write your pallas kernel here")


# speedup vs baseline: 1.1693x; 1.1693x over previous
import functools

import jax
import jax.numpy as jnp
from jax import lax
from jax.experimental import pallas as pl
from jax.experimental.pallas import tpu as pltpu
from jax.experimental.pallas import tpu_sc as plsc

B = 32
S = 128
E = 1024
F = 16
L = 64
D = 1024
H = 1024
A = 512
R = 1024
V = 32000

HI = jax.lax.Precision.HIGHEST

N = F * L
K = L
NCHUNK = N // 16


def _sc_facts_body(idx_hbm, w_hbm, fe_hbm, out_hbm,
                   idx64_v, w64_v, rows_v, acc_v, sem):
    b = lax.axis_index("c") * 16 + lax.axis_index("s")
    pltpu.sync_copy(idx_hbm.at[b], idx64_v)
    pltpu.sync_copy(w_hbm.at[b], w64_v)
    for c in range(K // 16):
        idx64_v[pl.ds(c * 16, 16)] = idx64_v[pl.ds(c * 16, 16)] + b * N
    pltpu.async_copy(fe_hbm.at[idx64_v], rows_v, sem).wait()

    wch = [w64_v[pl.ds(wc * 16, 16)] for wc in range(K // 16)]

    def acc_body(cc, _):
        acc = jnp.zeros((16,), jnp.float32)
        for wc in range(K // 16):
            for lane in range(16):
                j = wc * 16 + lane
                acc = acc + wch[wc][lane] * rows_v[j, pl.ds(cc * 16, 16)]
        acc_v[pl.ds(cc * 16, 16)] = acc
        return 0

    lax.fori_loop(0, D // 16, acc_body, 0)
    pltpu.sync_copy(acc_v, out_hbm.at[b])


def _sc_facts_select(sel_idx, sel_w, fe_flat_rows):
    mesh = plsc.VectorSubcoreMesh(core_axis_name="c", subcore_axis_name="s")
    kfn = functools.partial(
        pl.kernel, mesh=mesh,
        out_type=jax.ShapeDtypeStruct((B, D), jnp.float32),
        scratch_types=[
            pltpu.VMEM((K,), jnp.int32),
            pltpu.VMEM((K,), jnp.float32),
            pltpu.VMEM((K, D), jnp.float32),
            pltpu.VMEM((D,), jnp.float32),
            pltpu.SemaphoreType.DMA,
        ],
    )(_sc_facts_body)
    return kfn(sel_idx, sel_w, fe_flat_rows)


BQ = 4


def _q_att_kernel(bqe_ref, dh_ref, wqk_ref, wqh_ref, vq_ref, out_ref):
    bqe2 = bqe_ref[...].reshape(BQ * S, E)
    hq = jnp.dot(dh_ref[...].reshape(BQ, H), wqh_ref[...],
                 preferred_element_type=jnp.float32)
    t = jnp.dot(bqe2, wqk_ref[...],
                preferred_element_type=jnp.float32)
    th = jnp.tanh(t.reshape(BQ, S, A) + hq[:, None, :]).reshape(BQ * S, A)
    sv = lax.dot_general(vq_ref[...], th, (((1,), (1,)), ((), ())),
                         preferred_element_type=jnp.float32)
    rows = []
    for i in range(BQ):
        si = sv[:, i * S:(i + 1) * S]
        mi = jnp.max(si, axis=1, keepdims=True)
        ei = jnp.exp(si - mi)
        wi = ei / jnp.sum(ei, axis=1, keepdims=True)
        pieces = []
        if i:
            pieces.append(jnp.zeros((1, i * S), jnp.float32))
        pieces.append(wi)
        if i < BQ - 1:
            pieces.append(jnp.zeros((1, (BQ - 1 - i) * S), jnp.float32))
        rows.append(jnp.concatenate(pieces, axis=1))
    wblk = jnp.concatenate(rows, axis=0)
    q4 = jnp.dot(wblk, bqe2, preferred_element_type=jnp.float32)
    out_ref[...] = q4.reshape(BQ, 1, E)


def _facts_scores_kernel(fe_ref, dh_ref, wfk_ref, wfh_ref, vf_ref,
                         idx_ref, w_ref, scores_s):
    t = pl.program_id(0)

    @pl.when(t < B // BQ)
    def _score_step():
        fe2 = fe_ref[...].reshape(BQ * N, D)
        hf = jnp.dot(dh_ref[...].reshape(BQ, H), wfh_ref[...],
                     preferred_element_type=jnp.float32)
        tt = jnp.dot(fe2, wfk_ref[...],
                     preferred_element_type=jnp.float32)
        th = jnp.tanh(tt.reshape(BQ, N, A) + hf[:, None, :]).reshape(BQ * N, A)
        s = lax.dot_general(vf_ref[...], th, (((1,), (1,)), ((), ())),
                            preferred_element_type=jnp.float32)
        for i in range(BQ):
            scores_s[pl.ds(t * BQ + i, 1), :] = s[:, i * N:(i + 1) * N]

    @pl.when(t == B // BQ)
    def _select():
        s = scores_s[...]
        bits = lax.bitcast_convert_type(s, jnp.int32)
        keys = bits ^ ((bits >> 31) & jnp.int32(0x7FFFFFFF))

        def cnt_ge(cand):
            return jnp.sum(jnp.where(keys >= cand, jnp.float32(1), jnp.float32(0)),
                           axis=1, keepdims=True)

        cnt_pos = cnt_ge(jnp.zeros((B, 1), jnp.int32))
        t0 = jnp.where(cnt_pos >= K, jnp.int32(0), jnp.int32(-2147483648))

        def bit_body(j, tv):
            cand = tv + (jnp.int32(1) << (jnp.int32(30) - j))
            return jnp.where(cnt_ge(cand) >= K, cand, tv)

        thr = lax.fori_loop(0, 31, bit_body, t0)
        mask = (keys >= thr).astype(jnp.float32)

        jr = lax.broadcasted_iota(jnp.int32, (N, N), 0)
        jc = lax.broadcasted_iota(jnp.int32, (N, N), 1)
        triu = (jr <= jc).astype(jnp.float32)
        cum = jnp.dot(mask, triu, preferred_element_type=jnp.float32)
        keep = mask * (cum <= K).astype(jnp.float32)

        m_row = jnp.max(s, axis=1, keepdims=True)
        w_un = jnp.exp(s - m_row) * keep
        w_n = w_un / jnp.sum(w_un, axis=1, keepdims=True)

        jvec = lax.broadcasted_iota(jnp.int32, (B, N), 1).astype(jnp.float32)
        for p in range(1, K + 1):
            indp = keep * (cum == jnp.float32(p)).astype(jnp.float32)
            idx_col = jnp.sum(jvec * indp, axis=1, keepdims=True)
            w_col = jnp.sum(w_n * indp, axis=1, keepdims=True)
            idx_ref[:, pl.ds(p - 1, 1)] = idx_col.astype(jnp.int32)
            w_ref[:, pl.ds(p - 1, 1)] = w_col


VT = 3200


def _decoder_kernel(pte_ref, fatt_ref, qatt_ref, dh_ref, dc_ref,
                    wl_ref, ul_ref, bl_ref, wr_ref, br_ref, ur_ref,
                    bu_ref, vr_ref, bv_ref, wy_ref, by_ref,
                    h_ref, c_ref, logits_ref, z_scratch, mo_scratch):
    step = pl.program_id(0)

    @pl.when(step < 8)
    def _z_tile():
        wl = wl_ref[...]
        z = (jnp.dot(pte_ref[...], wl[0:E],
                     preferred_element_type=jnp.float32)
             + jnp.dot(fatt_ref[...], wl[E:E + D],
                       preferred_element_type=jnp.float32)
             + jnp.dot(qatt_ref[...], wl[E + D:E + D + E],
                       preferred_element_type=jnp.float32)
             + jnp.dot(dh_ref[...], ul_ref[...],
                       preferred_element_type=jnp.float32)
             + bl_ref[...])
        z_scratch[:, pl.ds(step * 512, 512)] = z

    @pl.when(step == 8)
    def _finish():
        z = z_scratch[...]
        i = jax.nn.sigmoid(z[:, 0:H])
        f = jax.nn.sigmoid(z[:, H:2 * H])
        g = jnp.tanh(z[:, 2 * H:3 * H])
        o = jax.nn.sigmoid(z[:, 3 * H:4 * H])
        c_new = f * dc_ref[...] + i * g
        h_new = o * jnp.tanh(c_new)
        c_ref[...] = c_new
        h_ref[...] = h_new
        ur = ur_ref[...]
        r = (jnp.dot(h_new, wr_ref[...],
                     preferred_element_type=jnp.float32)
             + jnp.dot(pte_ref[...], ur[0:E],
                       preferred_element_type=jnp.float32)
             + jnp.dot(fatt_ref[...], ur[E:E + D],
                       preferred_element_type=jnp.float32)
             + jnp.dot(qatt_ref[...], ur[E + D:E + D + E],
                       preferred_element_type=jnp.float32)
             + jnp.dot(qatt_ref[...], vr_ref[...],
                       preferred_element_type=jnp.float32)
             + br_ref[...] + bu_ref[...] + bv_ref[...])
        jr = lax.broadcasted_iota(jnp.int32, (R, R // 2), 0)
        ic = lax.broadcasted_iota(jnp.int32, (R, R // 2), 1)
        se = (jr == 2 * ic).astype(jnp.float32)
        so = (jr == 2 * ic + 1).astype(jnp.float32)
        r_even = jnp.dot(r, se, precision=HI, preferred_element_type=jnp.float32)
        r_odd = jnp.dot(r, so, precision=HI, preferred_element_type=jnp.float32)
        mo_scratch[...] = jnp.maximum(r_even, r_odd)

    @pl.when(step >= 9)
    def _vocab_tile():
        logits_ref[...] = jnp.dot(mo_scratch[...], wy_ref[...],
                                  preferred_element_type=jnp.float32) + by_ref[...]


def kernel(base_question_embeddings, facts_encodings, decoder_hidden, decoder_carry,
           previous_token_embedding, Wq_k, Wq_h, v_q, Wf_k, Wf_h, v_f,
           W_lstm, U_lstm, b_lstm, W_r, b_r, U_r, b_u, V_r, b_v, W_y, b_y):
    f32 = jnp.float32
    vq2 = v_q.reshape(1, A)
    vf2 = v_f.reshape(1, A)

    dh3 = decoder_hidden.reshape(B, 1, H)


    fe_flat = facts_encodings.reshape(B, F * L, D)
    sel_idx, sel_w = pl.pallas_call(
        _facts_scores_kernel,
        grid=(B // BQ + 1,),
        in_specs=[
            pl.BlockSpec((BQ, F * L, D),
                         lambda t: (jnp.minimum(t, B // BQ - 1), 0, 0)),
            pl.BlockSpec((BQ, 1, H),
                         lambda t: (jnp.minimum(t, B // BQ - 1), 0, 0)),
            pl.BlockSpec((D, A), lambda t: (0, 0)),
            pl.BlockSpec((H, A), lambda t: (0, 0)),
            pl.BlockSpec((1, A), lambda t: (0, 0)),
        ],
        out_specs=[
            pl.BlockSpec((B, K), lambda t: (0, 0)),
            pl.BlockSpec((B, K), lambda t: (0, 0)),
        ],
        out_shape=[
            jax.ShapeDtypeStruct((B, K), jnp.int32),
            jax.ShapeDtypeStruct((B, K), f32),
        ],
        scratch_shapes=[pltpu.VMEM((B, N), f32)],
    )(fe_flat, dh3, Wf_k, Wf_h, vf2)

    f_att = _sc_facts_select(sel_idx, sel_w, facts_encodings.reshape(B * N, D))

    q_att = pl.pallas_call(
        _q_att_kernel,
        grid=(B // BQ,),
        in_specs=[
            pl.BlockSpec((BQ, S, E), lambda b: (b, 0, 0)),
            pl.BlockSpec((BQ, 1, H), lambda b: (b, 0, 0)),
            pl.BlockSpec((E, A), lambda b: (0, 0)),
            pl.BlockSpec((H, A), lambda b: (0, 0)),
            pl.BlockSpec((1, A), lambda b: (0, 0)),
        ],
        out_specs=pl.BlockSpec((BQ, 1, E), lambda b: (b, 0, 0)),
        out_shape=jax.ShapeDtypeStruct((B, 1, E), f32),
    )(base_question_embeddings, dh3, Wq_k, Wq_h, vq2).reshape(B, E)

    h_new, c_new, logits = pl.pallas_call(
        _decoder_kernel,
        grid=(9 + V // VT,),
        in_specs=[
            pl.BlockSpec((B, E), lambda t: (0, 0)),
            pl.BlockSpec((B, D), lambda t: (0, 0)),
            pl.BlockSpec((B, E), lambda t: (0, 0)),
            pl.BlockSpec((B, H), lambda t: (0, 0)),
            pl.BlockSpec((B, H), lambda t: (0, 0)),
            pl.BlockSpec((E + D + E, 512), lambda t: (0, jnp.minimum(t, 7))),
            pl.BlockSpec((H, 512), lambda t: (0, jnp.minimum(t, 7))),
            pl.BlockSpec((1, 512), lambda t: (0, jnp.minimum(t, 7))),
            pl.BlockSpec((H, R), lambda t: (0, 0)),
            pl.BlockSpec((1, R), lambda t: (0, 0)),
            pl.BlockSpec((E + D + E, R), lambda t: (0, 0)),
            pl.BlockSpec((1, R), lambda t: (0, 0)),
            pl.BlockSpec((E, R), lambda t: (0, 0)),
            pl.BlockSpec((1, R), lambda t: (0, 0)),
            pl.BlockSpec((R // 2, VT),
                         lambda t: (0, jnp.maximum(t - 9, 0))),
            pl.BlockSpec((1, VT), lambda t: (0, jnp.maximum(t - 9, 0))),
        ],
        out_specs=[
            pl.BlockSpec((B, H), lambda t: (0, 0)),
            pl.BlockSpec((B, H), lambda t: (0, 0)),
            pl.BlockSpec((B, VT), lambda t: (0, jnp.maximum(t - 9, 0))),
        ],
        out_shape=[
            jax.ShapeDtypeStruct((B, H), f32),
            jax.ShapeDtypeStruct((B, H), f32),
            jax.ShapeDtypeStruct((B, V), f32),
        ],
        scratch_shapes=[pltpu.VMEM((B, 4 * H), f32),
                        pltpu.VMEM((B, R // 2), f32)],
    )(previous_token_embedding, f_att, q_att, decoder_hidden, decoder_carry,
      W_lstm, U_lstm, b_lstm.reshape(1, 4 * H), W_r, b_r.reshape(1, R),
      U_r, b_u.reshape(1, R), V_r, b_v.reshape(1, R), W_y, b_y.reshape(1, V))

    return (logits, h_new, c_new)
